# Initial kernel scaffold; baseline (speedup 1.0000x reference)
import jax
import jax.numpy as jnp
from jax.experimental import pallas as pl


def kernel(x, in_map, out_map, kernel):
    raise NotImplementedError("write your pallas kernel here")


# speedup vs baseline: 1.6011x; 1.6011x over previous
import functools

import jax
import jax.numpy as jnp
from jax import lax
from jax.experimental import pallas as pl
from jax.experimental.pallas import tpu as pltpu
from jax.experimental.pallas import tpu_sc as plsc

N_IN_ROWS = 100000
N_OUT_ROWS = 200000
NK = 27
NP = 20000
C = 128
KP = NK * NP

NCORE = 2
NSUB = 16
NW = NCORE * NSUB

ROWS_PER_W = 16896
KP_PAD = NW * ROWS_PER_W
GCHUNK = 512
NGCH = ROWS_PER_W // GCHUNK
IDX_ROWS_W = ROWS_PER_W // 128

MM_BLK = 2000

SCHUNK = 512
MS_PER_TILE = KP_PAD // NSUB
NSCH = MS_PER_TILE // SCHUNK
CS = 16
NSLAB = C // CS
HALF = N_OUT_ROWS // 2
TRASH = 1024
SLAB_R = HALF + TRASH
ZROWS = SLAB_R // NSUB
EXP_R = HALF // NSUB

_MESH = plsc.VectorSubcoreMesh(core_axis_name="c", subcore_axis_name="s")
_SC_PARAMS = pltpu.CompilerParams(use_tc_tiling_on_sc=False)


def _gather_body(x_hbm, im_hbm, g_hbm, idx_v, rows_v, sem):
    wid = lax.axis_index("s") * NCORE + lax.axis_index("c")
    pltpu.sync_copy(im_hbm.at[pl.ds(wid * IDX_ROWS_W, IDX_ROWS_W)], idx_v)
    base = wid * ROWS_PER_W

    def chunk(ci, carry):
        descs = []
        for j in range(GCHUNK // 128):
            d = pltpu.async_copy(
                x_hbm.at[idx_v.at[ci * (GCHUNK // 128) + j]],
                rows_v.at[pl.ds(j * 128, 128)],
                sem,
            )
            descs.append(d)
        for d in descs:
            d.wait()
        pltpu.sync_copy(rows_v, g_hbm.at[pl.ds(base + ci * GCHUNK, GCHUNK)])
        return carry

    lax.fori_loop(0, NGCH, chunk, 0)


@functools.partial(
    pl.kernel,
    out_type=jax.ShapeDtypeStruct((KP_PAD, C), jnp.float32),
    mesh=_MESH,
    scratch_types=[
        pltpu.VMEM((IDX_ROWS_W, 128), jnp.int32),
        pltpu.VMEM((GCHUNK, C), jnp.float32),
        pltpu.SemaphoreType.DMA,
    ],
    compiler_params=_SC_PARAMS,
)
def _gather_call(x_hbm, im_hbm, g_hbm, idx_v, rows_v, sem):
    _gather_body(x_hbm, im_hbm, g_hbm, idx_v, rows_v, sem)


def _mm_body(g_ref, w_ref, o_ref):
    o_ref[...] = jnp.dot(g_ref[...], w_ref[0], preferred_element_type=jnp.float32)


def _matmul(g, w):
    return pl.pallas_call(
        _mm_body,
        grid=(NK, NP // MM_BLK),
        in_specs=[
            pl.BlockSpec((MM_BLK, C), lambda k, b: (k * (NP // MM_BLK) + b, 0)),
            pl.BlockSpec((1, C, C), lambda k, b: (k, 0, 0)),
        ],
        out_specs=pl.BlockSpec((MM_BLK, C), lambda k, b: (k * (NP // MM_BLK) + b, 0)),
        out_shape=jax.ShapeDtypeStruct((KP_PAD, C), jnp.float32),
    )(g, w)


def _scatter_body(m_hbm, om_hbm, out_hbm, slab, data_v, idxr_v, idx_v, zero_v):
    cid = lax.axis_index("c")
    sid = lax.axis_index("s")
    rbase = cid * HALF

    def zfill(i, carry):
        zero_v[i, :] = jnp.zeros((CS,), jnp.float32)
        return carry

    lax.fori_loop(0, SCHUNK, zfill, 0)
    zfull = ZROWS // SCHUNK
    zrem = ZROWS - zfull * SCHUNK

    for s in range(NSLAB):
        c0 = s * CS
        for zi in range(zfull):
            pltpu.sync_copy(zero_v, slab.at[pl.ds(sid * ZROWS + zi * SCHUNK, SCHUNK)])
        pltpu.sync_copy(
            zero_v.at[pl.ds(0, zrem)],
            slab.at[pl.ds(sid * ZROWS + zfull * SCHUNK, zrem)],
        )
        plsc.subcore_barrier()

        def chunk(ci, carry):
            start = sid * MS_PER_TILE + ci * SCHUNK
            orow = sid * (MS_PER_TILE // 128) + ci * (SCHUNK // 128)
            pltpu.sync_copy(m_hbm.at[pl.ds(start, SCHUNK), pl.ds(c0, CS)], data_v)
            pltpu.sync_copy(om_hbm.at[pl.ds(orow, SCHUNK // 128)], idxr_v)
            for j in range(SCHUNK // 128):
                for i in range(128 // 16):
                    v = idxr_v[j, pl.ds(i * 16, 16)]
                    r = v - rbase
                    ok = (r >= 0) & (r < HALF)
                    t = HALF + (v & (TRASH - 1))
                    idx_v[j, pl.ds(i * 16, 16)] = jnp.where(ok, r, t)
            for j in range(SCHUNK // 128):
                pltpu.sync_copy(
                    data_v.at[pl.ds(j * 128, 128)],
                    slab.at[idx_v.at[j]],
                    add=True,
                )
            return carry

        lax.fori_loop(0, NSCH, chunk, 0)
        plsc.subcore_barrier()
        pltpu.sync_copy(
            slab.at[pl.ds(sid * EXP_R, EXP_R)],
            out_hbm.at[pl.ds(rbase + sid * EXP_R, EXP_R), pl.ds(c0, CS)],
        )
        plsc.subcore_barrier()


@functools.partial(
    pl.kernel,
    out_type=jax.ShapeDtypeStruct((N_OUT_ROWS, C), jnp.float32),
    mesh=_MESH,
    scratch_types=[
        pltpu.VMEM_SHARED((SLAB_R, CS), jnp.float32),
        pltpu.VMEM((SCHUNK, CS), jnp.float32),
        pltpu.VMEM((SCHUNK // 128, 128), jnp.int32),
        pltpu.VMEM((SCHUNK // 128, 128), jnp.int32),
        pltpu.VMEM((SCHUNK, CS), jnp.float32),
    ],
    compiler_params=_SC_PARAMS,
)
def _scatter_call(m_hbm, om_hbm, out_hbm, slab, data_v, idxr_v, idx_v, zero_v):
    _scatter_body(m_hbm, om_hbm, out_hbm, slab, data_v, idxr_v, idx_v, zero_v)


def kernel(x, in_map, out_map, kernel):
    w = kernel
    pad = KP_PAD - KP
    im = in_map.reshape(-1).astype(jnp.int32)
    om = out_map.reshape(-1).astype(jnp.int32)
    pad_in = (jnp.arange(pad, dtype=jnp.int32) * 149) % N_IN_ROWS
    pad_out = N_OUT_ROWS + jnp.arange(pad, dtype=jnp.int32)
    im_p = jnp.concatenate([im, pad_in]).reshape(KP_PAD // 128, 128)
    om_p = jnp.concatenate([om, pad_out]).reshape(KP_PAD // 128, 128)

    g = _gather_call(x, im_p)
    msgs = _matmul(g, w)
    return _scatter_call(msgs, om_p)


# speedup vs baseline: 2.4349x; 1.5207x over previous
import functools

import jax
import jax.numpy as jnp
from jax import lax
from jax.experimental import pallas as pl
from jax.experimental.pallas import tpu as pltpu
from jax.experimental.pallas import tpu_sc as plsc

N_IN_ROWS = 100000
N_OUT_ROWS = 200000
NK = 27
NP = 20000
C = 128
KP = NK * NP

NCORE = 2
NSUB = 16
NW = NCORE * NSUB

ROWS_PER_W = 16896
KP_PAD = NW * ROWS_PER_W
GCHUNK = 512
NGCH = ROWS_PER_W // GCHUNK
IDX_ROWS_W = ROWS_PER_W // 128

MM_BLK = 2000

SCHUNK = 512
MS_PER_TILE = KP_PAD // NSUB
NSCH = MS_PER_TILE // SCHUNK
CS = 16
NSLAB = C // CS
HALF = N_OUT_ROWS // 2
TRASH = 1024
SLAB_R = HALF + TRASH
ZROWS = SLAB_R // NSUB
EXP_R = HALF // NSUB

_MESH = plsc.VectorSubcoreMesh(core_axis_name="c", subcore_axis_name="s")
_SC_PARAMS = pltpu.CompilerParams(use_tc_tiling_on_sc=False)


def _gather_body(x_hbm, im_hbm, g_hbm, idx_v, rows_v, sem):
    wid = lax.axis_index("s") * NCORE + lax.axis_index("c")
    pltpu.sync_copy(im_hbm.at[pl.ds(wid * IDX_ROWS_W, IDX_ROWS_W)], idx_v)
    base = wid * ROWS_PER_W

    def chunk(ci, carry):
        descs = []
        for j in range(GCHUNK // 128):
            d = pltpu.async_copy(
                x_hbm.at[idx_v.at[ci * (GCHUNK // 128) + j]],
                rows_v.at[pl.ds(j * 128, 128)],
                sem,
            )
            descs.append(d)
        for d in descs:
            d.wait()
        pltpu.sync_copy(rows_v, g_hbm.at[pl.ds(base + ci * GCHUNK, GCHUNK)])
        return carry

    lax.fori_loop(0, NGCH, chunk, 0)


@functools.partial(
    pl.kernel,
    out_type=jax.ShapeDtypeStruct((KP_PAD, C), jnp.float32),
    mesh=_MESH,
    scratch_types=[
        pltpu.VMEM((IDX_ROWS_W, 128), jnp.int32),
        pltpu.VMEM((GCHUNK, C), jnp.float32),
        pltpu.SemaphoreType.DMA,
    ],
    compiler_params=_SC_PARAMS,
)
def _gather_call(x_hbm, im_hbm, g_hbm, idx_v, rows_v, sem):
    _gather_body(x_hbm, im_hbm, g_hbm, idx_v, rows_v, sem)


def _mm_body(g_ref, w_ref, o_ref):
    o_ref[...] = jnp.dot(g_ref[...], w_ref[0], preferred_element_type=jnp.float32)


def _matmul(g, w):
    return pl.pallas_call(
        _mm_body,
        grid=(NK, NP // MM_BLK),
        in_specs=[
            pl.BlockSpec((MM_BLK, C), lambda k, b: (k * (NP // MM_BLK) + b, 0)),
            pl.BlockSpec((1, C, C), lambda k, b: (k, 0, 0)),
        ],
        out_specs=pl.BlockSpec((MM_BLK, C), lambda k, b: (k * (NP // MM_BLK) + b, 0)),
        out_shape=jax.ShapeDtypeStruct((KP_PAD, C), jnp.float32),
    )(g, w)


def _scatter_body(m_hbm, om_hbm, out_hbm, slab, data_a, data_b, idxr_a, idxr_b,
                  idx_v, zero_v, sem_a, sem_b):
    cid = lax.axis_index("c")
    sid = lax.axis_index("s")
    rbase = cid * HALF
    irows = SCHUNK // 128

    def zfill(i, carry):
        zero_v[i, :] = jnp.zeros((CS,), jnp.float32)
        return carry

    lax.fori_loop(0, SCHUNK, zfill, 0)
    zfull = ZROWS // SCHUNK
    zrem = ZROWS - zfull * SCHUNK

    def issue(ci, c0, data_v, idxr_v, sem):
        start = sid * MS_PER_TILE + ci * SCHUNK
        orow = sid * (MS_PER_TILE // 128) + ci * irows
        pltpu.async_copy(m_hbm.at[pl.ds(start, SCHUNK), pl.ds(c0, CS)], data_v, sem)
        pltpu.async_copy(om_hbm.at[pl.ds(orow, irows)], idxr_v, sem)

    def drain(c0, data_v, idxr_v, sem):
        pltpu.make_async_copy(
            m_hbm.at[pl.ds(0, SCHUNK), pl.ds(c0, CS)], data_v, sem).wait()
        pltpu.make_async_copy(om_hbm.at[pl.ds(0, irows)], idxr_v, sem).wait()

    def process(data_v, idxr_v):
        for j in range(irows):
            for i in range(128 // 16):
                v = idxr_v[j, pl.ds(i * 16, 16)]
                r = v - rbase
                ok = (r >= 0) & (r < HALF)
                t = HALF + (v & (TRASH - 1))
                idx_v[j, pl.ds(i * 16, 16)] = jnp.where(ok, r, t)
        for j in range(irows):
            pltpu.sync_copy(
                data_v.at[pl.ds(j * 128, 128)],
                slab.at[idx_v.at[j]],
                add=True,
            )

    for s in range(NSLAB):
        c0 = s * CS
        for zi in range(zfull):
            pltpu.sync_copy(zero_v, slab.at[pl.ds(sid * ZROWS + zi * SCHUNK, SCHUNK)])
        pltpu.sync_copy(
            zero_v.at[pl.ds(0, zrem)],
            slab.at[pl.ds(sid * ZROWS + zfull * SCHUNK, zrem)],
        )
        plsc.subcore_barrier()

        issue(0, c0, data_a, idxr_a, sem_a)

        def pair(i, carry):
            issue(2 * i + 1, c0, data_b, idxr_b, sem_b)
            drain(c0, data_a, idxr_a, sem_a)
            process(data_a, idxr_a)

            @pl.when(i < NSCH // 2 - 1)
            def _():
                issue(2 * i + 2, c0, data_a, idxr_a, sem_a)

            drain(c0, data_b, idxr_b, sem_b)
            process(data_b, idxr_b)
            return carry

        lax.fori_loop(0, NSCH // 2, pair, 0)
        plsc.subcore_barrier()
        pltpu.sync_copy(
            slab.at[pl.ds(sid * EXP_R, EXP_R)],
            out_hbm.at[pl.ds(rbase + sid * EXP_R, EXP_R), pl.ds(c0, CS)],
        )
        plsc.subcore_barrier()


@functools.partial(
    pl.kernel,
    out_type=jax.ShapeDtypeStruct((N_OUT_ROWS, C), jnp.float32),
    mesh=_MESH,
    scratch_types=[
        pltpu.VMEM_SHARED((SLAB_R, CS), jnp.float32),
        pltpu.VMEM((SCHUNK, CS), jnp.float32),
        pltpu.VMEM((SCHUNK, CS), jnp.float32),
        pltpu.VMEM((SCHUNK // 128, 128), jnp.int32),
        pltpu.VMEM((SCHUNK // 128, 128), jnp.int32),
        pltpu.VMEM((SCHUNK // 128, 128), jnp.int32),
        pltpu.VMEM((SCHUNK, CS), jnp.float32),
        pltpu.SemaphoreType.DMA,
        pltpu.SemaphoreType.DMA,
    ],
    compiler_params=_SC_PARAMS,
)
def _scatter_call(m_hbm, om_hbm, out_hbm, slab, data_a, data_b, idxr_a, idxr_b,
                  idx_v, zero_v, sem_a, sem_b):
    _scatter_body(m_hbm, om_hbm, out_hbm, slab, data_a, data_b, idxr_a, idxr_b,
                  idx_v, zero_v, sem_a, sem_b)


def kernel(x, in_map, out_map, kernel):
    w = kernel
    pad = KP_PAD - KP
    im = in_map.reshape(-1).astype(jnp.int32)
    om = out_map.reshape(-1).astype(jnp.int32)
    pad_in = (jnp.arange(pad, dtype=jnp.int32) * 149) % N_IN_ROWS
    pad_out = N_OUT_ROWS + jnp.arange(pad, dtype=jnp.int32)
    im_p = jnp.concatenate([im, pad_in]).reshape(KP_PAD // 128, 128)
    om_p = jnp.concatenate([om, pad_out]).reshape(KP_PAD // 128, 128)

    g = _gather_call(x, im_p)
    msgs = _matmul(g, w)
    return _scatter_call(msgs, om_p)


# speedup vs baseline: 2.5446x; 1.0451x over previous
import functools

import jax
import jax.numpy as jnp
from jax import lax
from jax.experimental import pallas as pl
from jax.experimental.pallas import tpu as pltpu
from jax.experimental.pallas import tpu_sc as plsc

N_IN_ROWS = 100000
N_OUT_ROWS = 200000
NK = 27
NP = 20000
C = 128
KP = NK * NP

NCORE = 2
NSUB = 16
NW = NCORE * NSUB

ROWS_PER_W = 16896
KP_PAD = NW * ROWS_PER_W
GCHUNK = 384
NGCH = ROWS_PER_W // GCHUNK
GSTREAMS = GCHUNK // 128
IDX_ROWS_W = ROWS_PER_W // 128

MM_BLK = 2000

SCHUNK = 512
MS_PER_TILE = KP_PAD // NSUB
NSCH = MS_PER_TILE // SCHUNK
CS = 16
NSLAB = C // CS
HALF = N_OUT_ROWS // 2
TRASH = 1024
SLAB_R = HALF + TRASH
ZROWS = SLAB_R // NSUB
EXP_R = HALF // NSUB

_MESH = plsc.VectorSubcoreMesh(core_axis_name="c", subcore_axis_name="s")
_SC_PARAMS = pltpu.CompilerParams(use_tc_tiling_on_sc=False)


def _gather_body(x_hbm, im_hbm, g_hbm, idx_v, rows_a, rows_b,
                 sem_ga, sem_gb, sem_wa, sem_wb):
    wid = lax.axis_index("s") * NCORE + lax.axis_index("c")
    pltpu.sync_copy(im_hbm.at[pl.ds(wid * IDX_ROWS_W, IDX_ROWS_W)], idx_v)
    base = wid * ROWS_PER_W

    def issue_g(ci, rows_v, sem):
        for j in range(GSTREAMS):
            pltpu.async_copy(
                x_hbm.at[idx_v.at[ci * GSTREAMS + j]],
                rows_v.at[pl.ds(j * 128, 128)],
                sem,
            )

    def drain_g(rows_v, sem):
        for j in range(GSTREAMS):
            pltpu.make_async_copy(
                x_hbm.at[pl.ds(0, 128)],
                rows_v.at[pl.ds(j * 128, 128)],
                sem,
            ).wait()

    def issue_w(ci, rows_v, sem):
        pltpu.async_copy(rows_v, g_hbm.at[pl.ds(base + ci * GCHUNK, GCHUNK)], sem)

    def drain_w(rows_v, sem):
        pltpu.make_async_copy(rows_v, g_hbm.at[pl.ds(0, GCHUNK)], sem).wait()

    issue_g(0, rows_a, sem_ga)

    def pair(i, carry):
        c = 2 * i
        drain_g(rows_a, sem_ga)

        @pl.when(i > 0)
        def _():
            drain_w(rows_b, sem_wb)

        issue_g(c + 1, rows_b, sem_gb)
        issue_w(c, rows_a, sem_wa)
        drain_g(rows_b, sem_gb)
        drain_w(rows_a, sem_wa)

        @pl.when(i < NGCH // 2 - 1)
        def _():
            issue_g(c + 2, rows_a, sem_ga)

        issue_w(c + 1, rows_b, sem_wb)
        return carry

    lax.fori_loop(0, NGCH // 2, pair, 0)
    drain_w(rows_b, sem_wb)


@functools.partial(
    pl.kernel,
    out_type=jax.ShapeDtypeStruct((KP_PAD, C), jnp.float32),
    mesh=_MESH,
    scratch_types=[
        pltpu.VMEM((IDX_ROWS_W, 128), jnp.int32),
        pltpu.VMEM((GCHUNK, C), jnp.float32),
        pltpu.VMEM((GCHUNK, C), jnp.float32),
        pltpu.SemaphoreType.DMA,
        pltpu.SemaphoreType.DMA,
        pltpu.SemaphoreType.DMA,
        pltpu.SemaphoreType.DMA,
    ],
    compiler_params=_SC_PARAMS,
)
def _gather_call(x_hbm, im_hbm, g_hbm, idx_v, rows_a, rows_b,
                 sem_ga, sem_gb, sem_wa, sem_wb):
    _gather_body(x_hbm, im_hbm, g_hbm, idx_v, rows_a, rows_b,
                 sem_ga, sem_gb, sem_wa, sem_wb)


def _mm_body(g_ref, w_ref, o_ref):
    o_ref[...] = jnp.dot(g_ref[...], w_ref[0], preferred_element_type=jnp.float32)


def _matmul(g, w):
    return pl.pallas_call(
        _mm_body,
        grid=(NK, NP // MM_BLK),
        in_specs=[
            pl.BlockSpec((MM_BLK, C), lambda k, b: (k * (NP // MM_BLK) + b, 0)),
            pl.BlockSpec((1, C, C), lambda k, b: (k, 0, 0)),
        ],
        out_specs=pl.BlockSpec((MM_BLK, C), lambda k, b: (k * (NP // MM_BLK) + b, 0)),
        out_shape=jax.ShapeDtypeStruct((KP_PAD, C), jnp.float32),
    )(g, w)


def _scatter_body(m_hbm, om_hbm, out_hbm, slab, data_a, data_b, idxr_a, idxr_b,
                  idx_a, idx_b, zero_v, sem_a, sem_b, sem_add_a, sem_add_b):
    cid = lax.axis_index("c")
    sid = lax.axis_index("s")
    rbase = cid * HALF
    irows = SCHUNK // 128

    def zfill(i, carry):
        zero_v[i, :] = jnp.zeros((CS,), jnp.float32)
        return carry

    lax.fori_loop(0, SCHUNK, zfill, 0)
    zfull = ZROWS // SCHUNK
    zrem = ZROWS - zfull * SCHUNK

    def issue(ci, c0, data_v, idxr_v, sem):
        start = sid * MS_PER_TILE + ci * SCHUNK
        orow = sid * (MS_PER_TILE // 128) + ci * irows
        pltpu.async_copy(m_hbm.at[pl.ds(start, SCHUNK), pl.ds(c0, CS)], data_v, sem)
        pltpu.async_copy(om_hbm.at[pl.ds(orow, irows)], idxr_v, sem)

    def drain(c0, data_v, idxr_v, sem):
        pltpu.make_async_copy(
            m_hbm.at[pl.ds(0, SCHUNK), pl.ds(c0, CS)], data_v, sem).wait()
        pltpu.make_async_copy(om_hbm.at[pl.ds(0, irows)], idxr_v, sem).wait()

    def process(data_v, idxr_v, idx_v, sem_add):
        for j in range(irows):
            for i in range(128 // 16):
                v = idxr_v[j, pl.ds(i * 16, 16)]
                r = v - rbase
                ok = (r >= 0) & (r < HALF)
                t = HALF + (v & (TRASH - 1))
                idx_v[j, pl.ds(i * 16, 16)] = jnp.where(ok, r, t)
        descs = []
        for j in range(irows):
            descs.append(pltpu.async_copy(
                data_v.at[pl.ds(j * 128, 128)],
                slab.at[idx_v.at[j]],
                sem_add,
                add=True,
            ))
        for d in descs:
            d.wait()

    for s in range(NSLAB):
        c0 = s * CS
        for zi in range(zfull):
            pltpu.sync_copy(zero_v, slab.at[pl.ds(sid * ZROWS + zi * SCHUNK, SCHUNK)])
        pltpu.sync_copy(
            zero_v.at[pl.ds(0, zrem)],
            slab.at[pl.ds(sid * ZROWS + zfull * SCHUNK, zrem)],
        )
        plsc.subcore_barrier()

        issue(0, c0, data_a, idxr_a, sem_a)

        def pair(i, carry):
            issue(2 * i + 1, c0, data_b, idxr_b, sem_b)
            drain(c0, data_a, idxr_a, sem_a)
            process(data_a, idxr_a, idx_a, sem_add_a)

            @pl.when(i < NSCH // 2 - 1)
            def _():
                issue(2 * i + 2, c0, data_a, idxr_a, sem_a)

            drain(c0, data_b, idxr_b, sem_b)
            process(data_b, idxr_b, idx_b, sem_add_b)
            return carry

        lax.fori_loop(0, NSCH // 2, pair, 0)
        plsc.subcore_barrier()
        pltpu.sync_copy(
            slab.at[pl.ds(sid * EXP_R, EXP_R)],
            out_hbm.at[pl.ds(rbase + sid * EXP_R, EXP_R), pl.ds(c0, CS)],
        )
        plsc.subcore_barrier()


@functools.partial(
    pl.kernel,
    out_type=jax.ShapeDtypeStruct((N_OUT_ROWS, C), jnp.float32),
    mesh=_MESH,
    scratch_types=[
        pltpu.VMEM_SHARED((SLAB_R, CS), jnp.float32),
        pltpu.VMEM((SCHUNK, CS), jnp.float32),
        pltpu.VMEM((SCHUNK, CS), jnp.float32),
        pltpu.VMEM((SCHUNK // 128, 128), jnp.int32),
        pltpu.VMEM((SCHUNK // 128, 128), jnp.int32),
        pltpu.VMEM((SCHUNK // 128, 128), jnp.int32),
        pltpu.VMEM((SCHUNK // 128, 128), jnp.int32),
        pltpu.VMEM((SCHUNK, CS), jnp.float32),
        pltpu.SemaphoreType.DMA,
        pltpu.SemaphoreType.DMA,
        pltpu.SemaphoreType.DMA,
        pltpu.SemaphoreType.DMA,
    ],
    compiler_params=_SC_PARAMS,
)
def _scatter_call(m_hbm, om_hbm, out_hbm, slab, data_a, data_b, idxr_a, idxr_b,
                  idx_a, idx_b, zero_v, sem_a, sem_b, sem_add_a, sem_add_b):
    _scatter_body(m_hbm, om_hbm, out_hbm, slab, data_a, data_b, idxr_a, idxr_b,
                  idx_a, idx_b, zero_v, sem_a, sem_b, sem_add_a, sem_add_b)


def kernel(x, in_map, out_map, kernel):
    w = kernel
    pad = KP_PAD - KP
    im = in_map.reshape(-1).astype(jnp.int32)
    om = out_map.reshape(-1).astype(jnp.int32)
    pad_in = (jnp.arange(pad, dtype=jnp.int32) * 149) % N_IN_ROWS
    pad_out = N_OUT_ROWS + jnp.arange(pad, dtype=jnp.int32)
    im_p = jnp.concatenate([im, pad_in]).reshape(KP_PAD // 128, 128)
    om_p = jnp.concatenate([om, pad_out]).reshape(KP_PAD // 128, 128)

    g = _gather_call(x, im_p)
    msgs = _matmul(g, w)
    return _scatter_call(msgs, om_p)
